# Initial kernel scaffold; baseline (speedup 1.0000x reference)
import jax
import jax.numpy as jnp
from jax.experimental import pallas as pl


def kernel(h, edge_index, W, a):
    raise NotImplementedError("write your pallas kernel here")


# speedup vs baseline: 26.9803x; 26.9803x over previous
import functools

import jax
import jax.numpy as jnp
from jax import lax
from jax.experimental import pallas as pl
from jax.experimental.pallas import tpu as pltpu
from jax.experimental.pallas import tpu_sc as plsc

N = 10000
IN_F = 128
OUT_F = 128
ALPHA = 0.2
E = 320000
DP = 144
NC = 2
NS = 16
NW = NC * NS
EC = E // NW
B = 80
NCHUNK = EC // B
RT = N // NS
RZ = 125


def _tc1_body(h_ref, w_ref, a2_ref, u_ref, g16_ref):
    wh = jax.nn.leaky_relu(
        jnp.dot(h_ref[...], w_ref[...], preferred_element_type=jnp.float32),
        negative_slope=ALPHA)
    e2 = jnp.sum(wh * a2_ref[...], axis=1, keepdims=True)
    g = jnp.exp(e2 - jnp.max(e2))
    u_ref[...] = wh * g
    lane = lax.broadcasted_iota(jnp.int32, (N, 16), 1)
    g16_ref[...] = jnp.where(lane == 0, g, 0.0)


def _tc1(h, w, a2):
    return pl.pallas_call(
        _tc1_body,
        out_shape=(
            jax.ShapeDtypeStruct((N, OUT_F), jnp.float32),
            jax.ShapeDtypeStruct((N, 16), jnp.float32),
        ),
    )(h, w, a2)


def _sc_body(u_hbm, src_hbm, dst_hbm, z_hbm, out_hbm,
             src_v, dst_v, rows_v, stage_v, acc_sh, sem):
    cid = lax.axis_index("c")
    sid = lax.axis_index("s")
    wid = sid * NC + cid

    pltpu.sync_copy(z_hbm, stage_v)
    for k in range(RT // RZ):
        pltpu.sync_copy(stage_v, acc_sh.at[pl.ds(sid * RT + k * RZ, RZ)])
    plsc.subcore_barrier()

    ebase = wid * EC

    def body(j, carry):
        base = ebase + j * B
        pltpu.sync_copy(src_hbm.at[pl.ds(base, B)], src_v)
        pltpu.sync_copy(dst_hbm.at[pl.ds(base, B)], dst_v)
        pltpu.async_copy(u_hbm.at[dst_v], rows_v, sem).wait()
        pltpu.sync_copy(rows_v, acc_sh.at[src_v], add=True)
        return carry

    lax.fori_loop(0, NCHUNK, body, 0)
    plsc.subcore_barrier()

    for k in range(RT // RZ):
        r0 = sid * RT + k * RZ
        pltpu.sync_copy(acc_sh.at[pl.ds(r0, RZ)], stage_v)
        pltpu.sync_copy(stage_v, out_hbm.at[cid, pl.ds(r0, RZ)])


@functools.lru_cache(maxsize=None)
def _sc_agg():
    return pl.kernel(
        _sc_body,
        out_type=jax.ShapeDtypeStruct((NC, N, DP), jnp.float32),
        mesh=plsc.VectorSubcoreMesh(core_axis_name="c", subcore_axis_name="s"),
        compiler_params=pltpu.CompilerParams(use_tc_tiling_on_sc=False),
        scratch_types=[
            pltpu.VMEM((B,), jnp.int32),
            pltpu.VMEM((B,), jnp.int32),
            pltpu.VMEM((B, DP), jnp.float32),
            pltpu.VMEM((RZ, DP), jnp.float32),
            pltpu.VMEM_SHARED((N, DP), jnp.float32),
            pltpu.SemaphoreType.DMA,
        ],
    )


def _tc2_body(acc_ref, out_ref):
    a0 = acc_ref[0]
    a1 = acc_ref[1]
    num = a0[:, :OUT_F] + a1[:, :OUT_F]
    den = jnp.sum(a0[:, OUT_F:] + a1[:, OUT_F:], axis=1, keepdims=True)
    pos = den > 0.0
    hp = jnp.where(pos, num / jnp.where(pos, den, 1.0), 0.0)
    out_ref[...] = jnp.where(hp > 0.0, hp,
                             jnp.exp(jnp.minimum(hp, 0.0)) - 1.0)


def _tc2(acc):
    return pl.pallas_call(
        _tc2_body,
        out_shape=jax.ShapeDtypeStruct((N, OUT_F), jnp.float32),
    )(acc)


def kernel(h, edge_index, W, a):
    a2 = a[OUT_F:, 0][None, :]
    u, g16 = _tc1(h, W, a2)
    u144 = jnp.concatenate([u, g16], axis=1)
    src = edge_index[0]
    dst = edge_index[1]
    z = jnp.zeros((RZ, DP), jnp.float32)
    acc = _sc_agg()(u144, src, dst, z)
    return _tc2(acc)


# speedup vs baseline: 49.8473x; 1.8475x over previous
import functools

import jax
import jax.numpy as jnp
from jax import lax
from jax.experimental import pallas as pl
from jax.experimental.pallas import tpu as pltpu
from jax.experimental.pallas import tpu_sc as plsc

N = 10000
IN_F = 128
OUT_F = 128
ALPHA = 0.2
E = 320000
DP = 144
NC = 2
NS = 16
NW = NC * NS
EC = E // NW
B = 80
NCHUNK = EC // B
RT = N // NS


def _tc1_body(h_ref, w_ref, a2_ref, u_ref, g16_ref):
    wh = jax.nn.leaky_relu(
        jnp.dot(h_ref[...], w_ref[...], preferred_element_type=jnp.float32),
        negative_slope=ALPHA)
    e2 = jnp.sum(wh * a2_ref[...], axis=1, keepdims=True)
    g = jnp.exp(e2 - jnp.max(e2))
    u_ref[...] = wh * g
    lane = lax.broadcasted_iota(jnp.int32, (N, 16), 1)
    g16_ref[...] = jnp.where(lane == 0, g, 0.0)


def _tc1(h, w, a2):
    return pl.pallas_call(
        _tc1_body,
        out_shape=(
            jax.ShapeDtypeStruct((N, OUT_F), jnp.float32),
            jax.ShapeDtypeStruct((N, 16), jnp.float32),
        ),
    )(h, w, a2)


NBUF = 2


def _sc_body(u_hbm, src2_hbm, dst2_hbm, z_hbm, out_hbm,
             src_ring, dst_all, rows_all, acc_sh,
             sem_i, sem_g, sem_sc):
    cid = lax.axis_index("c")
    sid = lax.axis_index("s")
    wid = sid * NC + cid

    pltpu.sync_copy(z_hbm, rows_all.at[0])
    for k in range(RT // B):
        pltpu.sync_copy(rows_all.at[0], acc_sh.at[pl.ds(sid * RT + k * B, B)])
    rem = RT - (RT // B) * B
    if rem:
        pltpu.sync_copy(rows_all.at[0, pl.ds(0, rem)],
                        acc_sh.at[pl.ds(sid * RT + (RT // B) * B, rem)])
    pltpu.sync_copy(dst2_hbm.at[pl.ds(wid * NCHUNK, NCHUNK)], dst_all)
    plsc.subcore_barrier()

    def _fetch(j, buf):
        pltpu.async_copy(src2_hbm.at[wid * NCHUNK + j], src_ring.at[buf],
                         sem_i)
        pltpu.async_copy(u_hbm.at[dst_all.at[j]], rows_all.at[buf], sem_g)

    def _drain(sem, ref):
        pltpu.make_async_copy(u_hbm.at[pl.ds(0, B)], ref, sem).wait()

    _fetch(0, 0)

    def body(j, carry):
        nxt = j + 1

        @pl.when(nxt < NCHUNK)
        def _prefetch():
            @pl.when(nxt >= NBUF)
            def _free_buf():
                _drain(sem_sc, rows_all.at[0])
            _fetch(nxt, lax.rem(nxt, NBUF))

        _drain(sem_g, rows_all.at[0])
        _drain(sem_i, src_ring.at[0])
        buf = lax.rem(j, NBUF)
        pltpu.async_copy(rows_all.at[buf], acc_sh.at[src_ring.at[buf]],
                         sem_sc, add=True)
        return carry

    lax.fori_loop(0, NCHUNK, body, 0)
    for _ in range(NBUF):
        _drain(sem_sc, rows_all.at[0])
    plsc.subcore_barrier()

    for k in range(RT // B + (1 if RT % B else 0)):
        r0 = sid * RT + k * B
        w = min(B, RT - k * B)
        pltpu.sync_copy(acc_sh.at[pl.ds(r0, w)], rows_all.at[0, pl.ds(0, w)])
        pltpu.sync_copy(rows_all.at[0, pl.ds(0, w)],
                        out_hbm.at[cid, pl.ds(r0, w)])


@functools.lru_cache(maxsize=None)
def _sc_agg():
    return pl.kernel(
        _sc_body,
        out_type=jax.ShapeDtypeStruct((NC, N, DP), jnp.float32),
        mesh=plsc.VectorSubcoreMesh(core_axis_name="c", subcore_axis_name="s"),
        compiler_params=pltpu.CompilerParams(use_tc_tiling_on_sc=False),
        scratch_types=[
            pltpu.VMEM((NBUF, B), jnp.int32),
            pltpu.VMEM((NCHUNK, B), jnp.int32),
            pltpu.VMEM((NBUF, B, DP), jnp.float32),
            pltpu.VMEM_SHARED((N, DP), jnp.float32),
            pltpu.SemaphoreType.DMA,
            pltpu.SemaphoreType.DMA,
            pltpu.SemaphoreType.DMA,
        ],
    )


def _tc2_body(acc_ref, out_ref):
    a0 = acc_ref[0]
    a1 = acc_ref[1]
    num = a0[:, :OUT_F] + a1[:, :OUT_F]
    den = jnp.sum(a0[:, OUT_F:] + a1[:, OUT_F:], axis=1, keepdims=True)
    pos = den > 0.0
    hp = jnp.where(pos, num / jnp.where(pos, den, 1.0), 0.0)
    out_ref[...] = jnp.where(hp > 0.0, hp,
                             jnp.exp(jnp.minimum(hp, 0.0)) - 1.0)


def _tc2(acc):
    return pl.pallas_call(
        _tc2_body,
        out_shape=jax.ShapeDtypeStruct((N, OUT_F), jnp.float32),
    )(acc)


def kernel(h, edge_index, W, a):
    a2 = a[OUT_F:, 0][None, :]
    u, g16 = _tc1(h, W, a2)
    u144 = jnp.concatenate([u, g16], axis=1)
    src2 = edge_index[0].reshape(E // B, B)
    dst2 = edge_index[1].reshape(E // B, B)
    z = jnp.zeros((B, DP), jnp.float32)
    acc = _sc_agg()(u144, src2, dst2, z)
    return _tc2(acc)
